# Initial kernel scaffold; baseline (speedup 1.0000x reference)
import jax
import jax.numpy as jnp
from jax.experimental import pallas as pl


def kernel(source_node_id, target_node_id, target_x, edge_index, edge_label_index, Es, Et, Wlin, blin, Wl1_st, b1_st, Wr1_st, Wl1_ts, b1_ts, Wr1_ts, Wl2_st, b2_st, Wr2_st, Wl2_ts, b2_ts, Wr2_ts):
    raise NotImplementedError("write your pallas kernel here")


# speedup vs baseline: 1.7100x; 1.7100x over previous
import functools

import jax
import jax.numpy as jnp
from jax import lax
from jax.experimental import pallas as pl
from jax.experimental.pallas import tpu as pltpu
from jax.experimental.pallas import tpu_sc as plsc

NS = 50000
NT = 50000
E = 625000
L = 100000
H = 128
O = 64
DIN = 20
DINP = 24

C = 12800
NCH = 4
NCOVER = C * NCH
SENT = NCOVER
CACC = C + 16
NCHPS = NCH // 2
CSL = C // 16
ZSL = CACC // 16

NCT = 55296
DSL = NCT // 16

EBLK = 1024
NEB = 40
EPT = EBLK * NEB
EP = 16 * EPT

KI = 64
KR = 3
K = KI * KR

LBLK = 128
NLB = 25
LPT = LBLK * NLB
LP = 32 * LPT

BM = 1024
NPB = 49
NP = BM * NPB

_SC_PARAMS = pltpu.CompilerParams(needs_layout_passes=False)


@functools.lru_cache(maxsize=None)
def _mesh():
  return plsc.VectorSubcoreMesh(core_axis_name="c", subcore_axis_name="s",
                                num_cores=2, num_subcores=16)


@functools.lru_cache(maxsize=None)
def _make_degrees():
  scratch = [
      pltpu.VMEM((EBLK // 128, 128), jnp.int32),
      pltpu.VMEM((EBLK // 128, 128), jnp.int32),
      pltpu.VMEM((128,), jnp.float32),
      pltpu.VMEM((DSL,), jnp.float32),
      pltpu.SemaphoreType.DMA,
      pltpu.VMEM_SHARED((NCT,), jnp.float32),
      pltpu.VMEM_SHARED((NCT,), jnp.float32),
  ]
  out_type = [jax.ShapeDtypeStruct((2 * NCT,), jnp.float32),
              jax.ShapeDtypeStruct((2 * NCT,), jnp.float32)]

  def body(src2, dst2, onesd, zd, out_s, out_t,
           sblk, dblk, ones_v, cbuf, sem, cnt_s, cnt_t):
    cid = lax.axis_index("c")
    sid = lax.axis_index("s")
    erows = (sid * 2 + cid) * (EP // 32 // 128)

    pltpu.sync_copy(onesd, ones_v)
    pltpu.sync_copy(zd, cbuf)
    pltpu.sync_copy(cbuf, cnt_s.at[pl.ds(DSL * sid, DSL)])
    pltpu.sync_copy(cbuf, cnt_t.at[pl.ds(DSL * sid, DSL)])
    plsc.subcore_barrier()

    def block(b, _):
      pltpu.sync_copy(src2.at[pl.ds(erows + b * (EBLK // 128), EBLK // 128)],
                      sblk)
      pltpu.sync_copy(dst2.at[pl.ds(erows + b * (EBLK // 128), EBLK // 128)],
                      dblk)
      cps = []
      for r in range(EBLK // 128):
        cps.append(pltpu.async_copy(ones_v, cnt_s.at[sblk.at[r]], sem,
                                    add=True))
        cps.append(pltpu.async_copy(ones_v, cnt_t.at[dblk.at[r]], sem,
                                    add=True))
      for cp in cps:
        cp.wait()
      return 0

    lax.fori_loop(0, NEB // 2, block, 0)
    plsc.subcore_barrier()

    obase = pl.multiple_of(cid * NCT + DSL * sid, 128)
    pltpu.sync_copy(cnt_s.at[pl.ds(DSL * sid, DSL)], cbuf)
    pltpu.sync_copy(cbuf, out_s.at[pl.ds(obase, DSL)])
    pltpu.sync_copy(cnt_t.at[pl.ds(DSL * sid, DSL)], cbuf)
    pltpu.sync_copy(cbuf, out_t.at[pl.ds(obase, DSL)])

  return pl.kernel(body, out_type=out_type, mesh=_mesh(),
                   scratch_types=scratch, name="sc_degrees",
                   compiler_params=_SC_PARAMS)


def _degrees(src2, dst2, onesd, zd):
  return _make_degrees()(src2, dst2, onesd, zd)


@functools.lru_cache(maxsize=None)
def _make_agg():
  out_type = jax.ShapeDtypeStruct((NCOVER, H), jnp.float32)

  scratch = [
      pltpu.VMEM((EBLK,), jnp.int32),
      pltpu.VMEM((EBLK,), jnp.int32),
      pltpu.VMEM((KR, KI), jnp.int32),
      pltpu.VMEM((KR, KI), jnp.int32),
      pltpu.VMEM((K, H), jnp.float32),
      pltpu.SemaphoreType.DMA,
      pltpu.VMEM_SHARED((CACC, H), jnp.float32),
  ]

  def body(tab, gi, si, zrows, out, gblk, sblk, gidx, goff, rows, sem, acc):
    cid = lax.axis_index("c")
    sid = lax.axis_index("s")
    ebase = sid * EPT

    zi16 = jnp.zeros((16,), jnp.int32)
    doff16 = jnp.full((16,), C, jnp.int32)

    def reset_idx_bufs():
      for r in range(KR):
        for t in range(KI // 16):
          gidx[r, pl.ds(t * 16, 16)] = zi16
          goff[r, pl.ds(t * 16, 16)] = doff16

    reset_idx_bufs()

    def fire(_):
      cps = [pltpu.async_copy(tab.at[gidx.at[r]], rows.at[pl.ds(r * KI, KI)],
                              sem) for r in range(KR)]
      for cp in cps:
        cp.wait()
      for r in range(KR):
        pltpu.sync_copy(rows.at[pl.ds(r * KI, KI)], acc.at[goff.at[r]],
                        add=True)
      reset_idx_bufs()
      return jnp.int32(0)

    _ZCH = ((0, K), (K, K), (2 * K, K), (3 * K, K), (4 * K, ZSL - 4 * K))
    _WCH = ((0, K), (K, K), (2 * K, K), (3 * K, K), (4 * K, CSL - 4 * K))

    for kl in range(NCHPS):
      k = cid * NCHPS + kl
      lo = k * C

      pltpu.sync_copy(zrows.at[pl.ds(0, K)], rows)
      for off, n in _ZCH:
        pltpu.sync_copy(rows.at[pl.ds(0, n)],
                        acc.at[pl.ds(ZSL * sid + off, n)])
      plsc.subcore_barrier()

      def step(i, pos):
        gv = gblk[pl.ds(i * 16, 16)]
        sv = sblk[pl.ds(i * 16, 16)]
        m = (sv >= lo) & (sv < lo + C)
        mi = m.astype(jnp.int32)
        excl = plsc.cumsum(mi) - mi
        tgt = pos + excl
        r_i = lax.shift_right_logical(tgt, 6)
        c_i = jnp.bitwise_and(tgt, KI - 1)
        plsc.store_scatter(gidx, [r_i, c_i], gv, mask=m)
        plsc.store_scatter(goff, [r_i, c_i], sv - lo, mask=m)
        pos2 = pos + jnp.sum(mi)
        return lax.cond(pos2 > K - 16, fire, lambda p: p, pos2)

      def process_block(b, pos):
        pltpu.sync_copy(gi.at[pl.ds(ebase + b * EBLK, EBLK)], gblk)
        pltpu.sync_copy(si.at[pl.ds(ebase + b * EBLK, EBLK)], sblk)
        return lax.fori_loop(0, EBLK // 16, step, pos)

      pos = lax.fori_loop(0, NEB, process_block, jnp.int32(0))
      fire(pos)
      plsc.subcore_barrier()

      wbase = CSL * sid
      for off, n in _WCH:
        pltpu.sync_copy(acc.at[pl.ds(wbase + off, n)], rows.at[pl.ds(0, n)])
        pltpu.sync_copy(rows.at[pl.ds(0, n)],
                        out.at[pl.ds(k * C + wbase + off, n)])
      plsc.subcore_barrier()

  return pl.kernel(body, out_type=out_type, mesh=_mesh(),
                   scratch_types=scratch, name="sc_agg",
                   compiler_params=_SC_PARAMS)


def _agg(tab, gidx, sidx, zrows):
  return _make_agg()(tab, gidx, sidx, zrows)


def _sc_dot(o_s, o_t, e0, e1):
  scratch = [
      pltpu.VMEM((LBLK,), jnp.int32),
      pltpu.VMEM((LBLK,), jnp.int32),
      pltpu.VMEM((LBLK, H), jnp.float32),
      pltpu.VMEM((LBLK, H), jnp.float32),
      pltpu.VMEM((LBLK,), jnp.float32),
      pltpu.SemaphoreType.DMA,
  ]

  def body(os_hbm, ot_hbm, e0_hbm, e1_hbm, out, i0, i1, rs, rt, ob, sem):
    cid = lax.axis_index("c")
    sid = lax.axis_index("s")
    wid = sid * 2 + cid
    base = wid * LPT

    def block(b, _):
      off = base + b * LBLK
      pltpu.sync_copy(e0_hbm.at[pl.ds(off, LBLK)], i0)
      pltpu.sync_copy(e1_hbm.at[pl.ds(off, LBLK)], i1)
      cp0 = pltpu.async_copy(os_hbm.at[i0], rs, sem)
      cp1 = pltpu.async_copy(ot_hbm.at[i1], rt, sem)
      cp0.wait()
      cp1.wait()

      iota = lax.iota(jnp.int32, 16)

      def lab16(jj, _):
        rowi = jj * 16 + iota
        acc = jnp.zeros((16,), jnp.float32)
        for c in range(O):
          ci = jnp.full((16,), c, jnp.int32)
          acc = acc + (plsc.load_gather(rs, [rowi, ci])
                       * plsc.load_gather(rt, [rowi, ci]))
        ob[pl.ds(jj * 16, 16)] = acc
        return 0

      lax.fori_loop(0, LBLK // 16, lab16, 0)
      pltpu.sync_copy(ob, out.at[pl.ds(off, LBLK)])
      return 0

    lax.fori_loop(0, NLB, block, 0)

  f = pl.kernel(body, out_type=jax.ShapeDtypeStruct((LP,), jnp.float32),
                mesh=_mesh(), scratch_types=scratch, name="sc_dot",
                compiler_params=_SC_PARAMS)
  return f(o_s, o_t, e0, e1)


def _tc_xt(target_x, Wlin, blin, Et):
  def body(tx, w, b, et, o):
    o[...] = (jnp.dot(tx[...], w[...], preferred_element_type=jnp.float32)
              + b[...] + et[...])

  return pl.pallas_call(
      body, grid=(NPB,),
      in_specs=[
          pl.BlockSpec((BM, DINP), lambda i: (i, 0)),
          pl.BlockSpec((DINP, H), lambda i: (0, 0)),
          pl.BlockSpec((1, H), lambda i: (0, 0)),
          pl.BlockSpec((BM, H), lambda i: (i, 0)),
      ],
      out_specs=pl.BlockSpec((BM, H), lambda i: (i, 0)),
      out_shape=jax.ShapeDtypeStruct((NP, H), jnp.float32),
  )(target_x, Wlin, blin, Et)


def _tc_combine1(sums, cnt, x, Wl, Wr, b):
  def body(s, c, x_, wl, wr, b_, h_o):
    deg = jnp.clip(c[...], 1.0, None)
    a = s[...] / deg
    h_o[...] = jnp.maximum(
        jnp.dot(a, wl[...], preferred_element_type=jnp.float32)
        + jnp.dot(x_[...], wr[...], preferred_element_type=jnp.float32)
        + b_[...], 0.0)

  return pl.pallas_call(
      body, grid=(NPB,),
      in_specs=[
          pl.BlockSpec((BM, H), lambda i: (i, 0)),
          pl.BlockSpec((BM, 1), lambda i: (i, 0)),
          pl.BlockSpec((BM, H), lambda i: (i, 0)),
          pl.BlockSpec((H, H), lambda i: (0, 0)),
          pl.BlockSpec((H, H), lambda i: (0, 0)),
          pl.BlockSpec((1, H), lambda i: (0, 0)),
      ],
      out_specs=pl.BlockSpec((BM, H), lambda i: (i, 0)),
      out_shape=jax.ShapeDtypeStruct((NP, H), jnp.float32),
  )(sums, cnt, x, Wl, Wr, b)


def _tc_combine2(sums, cnt, h, Wl, Wr, b):
  def body(s, c, h_, wl, wr, b_, o_o):
    deg = jnp.clip(c[...], 1.0, None)
    a = s[...] / deg
    o = (jnp.dot(a, wl[...], preferred_element_type=jnp.float32)
         + jnp.dot(h_[...], wr[...], preferred_element_type=jnp.float32)
         + b_[...])
    o_o[...] = jnp.concatenate([o, jnp.zeros((BM, H - O), jnp.float32)],
                               axis=1)

  return pl.pallas_call(
      body, grid=(NPB,),
      in_specs=[
          pl.BlockSpec((BM, H), lambda i: (i, 0)),
          pl.BlockSpec((BM, 1), lambda i: (i, 0)),
          pl.BlockSpec((BM, H), lambda i: (i, 0)),
          pl.BlockSpec((H, O), lambda i: (0, 0)),
          pl.BlockSpec((H, O), lambda i: (0, 0)),
          pl.BlockSpec((1, O), lambda i: (0, 0)),
      ],
      out_specs=pl.BlockSpec((BM, H), lambda i: (i, 0)),
      out_shape=jax.ShapeDtypeStruct((NP, H), jnp.float32),
  )(sums, cnt, h, Wl, Wr, b)


def kernel(source_node_id, target_node_id, target_x, edge_index,
           edge_label_index, Es, Et, Wlin, blin,
           Wl1_st, b1_st, Wr1_st, Wl1_ts, b1_ts, Wr1_ts,
           Wl2_st, b2_st, Wr2_st, Wl2_ts, b2_ts, Wr2_ts):
  src = edge_index[0].astype(jnp.int32)
  dst = edge_index[1].astype(jnp.int32)
  epad = jnp.full((EP - E,), SENT, jnp.int32)
  src_p = jnp.concatenate([src, epad])
  dst_p = jnp.concatenate([dst, epad])
  src2 = src_p.reshape(EP // 128, 128)
  dst2 = dst_p.reshape(EP // 128, 128)
  e0 = jnp.concatenate([edge_label_index[0].astype(jnp.int32),
                        jnp.zeros((LP - L,), jnp.int32)])
  e1 = jnp.concatenate([edge_label_index[1].astype(jnp.int32),
                        jnp.zeros((LP - L,), jnp.int32)])

  txp = jnp.pad(target_x, ((0, 0), (0, DINP - DIN)))
  wlinp = jnp.pad(Wlin, ((0, DINP - DIN), (0, 0)))
  b_lin = blin.reshape(1, H)
  b1st = b1_st.reshape(1, H)
  b1ts = b1_ts.reshape(1, H)
  b2st = b2_st.reshape(1, O)
  b2ts = b2_ts.reshape(1, O)

  zrows = jnp.zeros((ZSL, H), jnp.float32)
  zd = jnp.zeros((DSL,), jnp.float32)
  onesd = jnp.ones((128,), jnp.float32)

  x_s = Es
  x_t = _tc_xt(txp, wlinp, b_lin, Et)

  cnt_s_p, cnt_t_p = _degrees(src2, dst2, onesd, zd)
  cnt_s2 = (cnt_s_p[:NCT] + cnt_s_p[NCT:])[:, None]
  cnt_t2 = (cnt_t_p[:NCT] + cnt_t_p[NCT:])[:, None]

  sum1_t = _agg(x_s, src_p, dst_p, zrows)
  sum1_s = _agg(x_t, dst_p, src_p, zrows)

  h_t = _tc_combine1(sum1_t, cnt_t2, x_t, Wl1_st, Wr1_st, b1st)
  h_s = _tc_combine1(sum1_s, cnt_s2, x_s, Wl1_ts, Wr1_ts, b1ts)

  sum2_t = _agg(h_s, src_p, dst_p, zrows)
  sum2_s = _agg(h_t, dst_p, src_p, zrows)

  o_t = _tc_combine2(sum2_t, cnt_t2, h_t, Wl2_st, Wr2_st, b2st)
  o_s = _tc_combine2(sum2_s, cnt_s2, h_s, Wl2_ts, Wr2_ts, b2ts)

  return _sc_dot(o_s, o_t, e0, e1)[:L]


# speedup vs baseline: 3.5126x; 2.0542x over previous
import functools

import jax
import jax.numpy as jnp
from jax import lax
from jax.experimental import pallas as pl
from jax.experimental.pallas import tpu as pltpu
from jax.experimental.pallas import tpu_sc as plsc

NS = 50000
NT = 50000
E = 625000
L = 100000
H = 128
O = 64
DIN = 20
DINP = 24

C = 8960
NCH = 6
NCOVER = C * NCH
SENT = NCOVER
CACC = C + 16
NCHPS = NCH // 2
CSL = C // 16
ZSL = CACC // 16

NCT = 55296
DSL = NCT // 16

EBLK = 1024
NEB = 40
EPT = EBLK * NEB
EP = 16 * EPT

KI = 128
KR = 3
K = KI * KR

LBLK = 128
NLB = 25
LPT = LBLK * NLB
LP = 32 * LPT

BM = 1024
NPB = 49
NP = BM * NPB

_SC_PARAMS = pltpu.CompilerParams(needs_layout_passes=False)


@functools.lru_cache(maxsize=None)
def _mesh():
  return plsc.VectorSubcoreMesh(core_axis_name="c", subcore_axis_name="s",
                                num_cores=2, num_subcores=16)


@functools.lru_cache(maxsize=None)
def _make_degrees():
  scratch = [
      pltpu.VMEM((EBLK // 128, 128), jnp.int32),
      pltpu.VMEM((EBLK // 128, 128), jnp.int32),
      pltpu.VMEM((128,), jnp.float32),
      pltpu.VMEM((DSL,), jnp.float32),
      pltpu.SemaphoreType.DMA,
      pltpu.VMEM_SHARED((NCT,), jnp.float32),
      pltpu.VMEM_SHARED((NCT,), jnp.float32),
  ]
  out_type = [jax.ShapeDtypeStruct((2 * NCT,), jnp.float32),
              jax.ShapeDtypeStruct((2 * NCT,), jnp.float32)]

  def body(src2, dst2, onesd, zd, out_s, out_t,
           sblk, dblk, ones_v, cbuf, sem, cnt_s, cnt_t):
    cid = lax.axis_index("c")
    sid = lax.axis_index("s")
    erows = (sid * 2 + cid) * (EP // 32 // 128)

    pltpu.sync_copy(onesd, ones_v)
    pltpu.sync_copy(zd, cbuf)
    pltpu.sync_copy(cbuf, cnt_s.at[pl.ds(DSL * sid, DSL)])
    pltpu.sync_copy(cbuf, cnt_t.at[pl.ds(DSL * sid, DSL)])
    plsc.subcore_barrier()

    def block(b, _):
      pltpu.sync_copy(src2.at[pl.ds(erows + b * (EBLK // 128), EBLK // 128)],
                      sblk)
      pltpu.sync_copy(dst2.at[pl.ds(erows + b * (EBLK // 128), EBLK // 128)],
                      dblk)
      cps = []
      for r in range(EBLK // 128):
        cps.append(pltpu.async_copy(ones_v, cnt_s.at[sblk.at[r]], sem,
                                    add=True))
        cps.append(pltpu.async_copy(ones_v, cnt_t.at[dblk.at[r]], sem,
                                    add=True))
      for cp in cps:
        cp.wait()
      return 0

    lax.fori_loop(0, NEB // 2, block, 0)
    plsc.subcore_barrier()

    obase = pl.multiple_of(cid * NCT + DSL * sid, 128)
    pltpu.sync_copy(cnt_s.at[pl.ds(DSL * sid, DSL)], cbuf)
    pltpu.sync_copy(cbuf, out_s.at[pl.ds(obase, DSL)])
    pltpu.sync_copy(cnt_t.at[pl.ds(DSL * sid, DSL)], cbuf)
    pltpu.sync_copy(cbuf, out_t.at[pl.ds(obase, DSL)])

  return pl.kernel(body, out_type=out_type, mesh=_mesh(),
                   scratch_types=scratch, name="sc_degrees",
                   compiler_params=_SC_PARAMS)


def _degrees(src2, dst2, onesd, zd):
  return _make_degrees()(src2, dst2, onesd, zd)


@functools.lru_cache(maxsize=None)
def _make_agg():
  out_type = jax.ShapeDtypeStruct((NCOVER, H), jnp.float32)

  scratch = [
      pltpu.VMEM((EBLK,), jnp.int32),
      pltpu.VMEM((EBLK,), jnp.int32),
      pltpu.VMEM((KR, KI), jnp.int32),
      pltpu.VMEM((KR, KI), jnp.int32),
      pltpu.VMEM((K, H), jnp.float32),
      pltpu.SemaphoreType.DMA,
      pltpu.SemaphoreType.DMA,
      pltpu.VMEM_SHARED((CACC, H), jnp.float32),
  ]

  def body(tab, gi, si, zrows, out, gblk, sblk, gidx, goff, rows, sem, sem2, acc):
    cid = lax.axis_index("c")
    sid = lax.axis_index("s")
    ebase = sid * EPT

    zi16 = jnp.zeros((16,), jnp.int32)
    doff16 = jnp.full((16,), C, jnp.int32)

    def reset_idx_bufs(also_gidx):
      for r in range(KR):
        for t in range(KI // 16):
          if also_gidx:
            gidx[r, pl.ds(t * 16, 16)] = zi16
          goff[r, pl.ds(t * 16, 16)] = doff16

    reset_idx_bufs(True)

    def fire(_):
      gcps = [pltpu.async_copy(tab.at[gidx.at[r]], rows.at[pl.ds(r * KI, KI)],
                               sem) for r in range(KR)]
      for cp in gcps:
        cp.wait()
      scps = [pltpu.async_copy(rows.at[pl.ds(r * KI, KI)], acc.at[goff.at[r]],
                               sem2, add=True) for r in range(KR)]
      for cp in scps:
        cp.wait()
      reset_idx_bufs(False)
      return jnp.int32(0)

    _ZCH = ((0, K), (K, ZSL - K))
    _WCH = ((0, K), (K, CSL - K))

    for kl in range(NCHPS):
      k = cid * NCHPS + kl
      lo = k * C

      pltpu.sync_copy(zrows.at[pl.ds(0, K)], rows)
      for off, n in _ZCH:
        pltpu.sync_copy(rows.at[pl.ds(0, n)],
                        acc.at[pl.ds(ZSL * sid + off, n)])
      plsc.subcore_barrier()

      def step(i, pos):
        gv = gblk[pl.ds(i * 16, 16)]
        sv = sblk[pl.ds(i * 16, 16)]
        m = (sv >= lo) & (sv < lo + C)
        mi = m.astype(jnp.int32)
        excl = plsc.cumsum(mi) - mi
        tgt = pos + excl
        r_i = lax.shift_right_logical(tgt, 7)
        c_i = jnp.bitwise_and(tgt, KI - 1)
        plsc.store_scatter(gidx, [r_i, c_i], gv, mask=m)
        plsc.store_scatter(goff, [r_i, c_i], sv - lo, mask=m)
        pos2 = pos + jnp.sum(mi)
        return lax.cond(pos2 > K - 16, fire, lambda p: p, pos2)

      def process_block(b, pos):
        pltpu.sync_copy(gi.at[pl.ds(ebase + b * EBLK, EBLK)], gblk)
        pltpu.sync_copy(si.at[pl.ds(ebase + b * EBLK, EBLK)], sblk)
        return lax.fori_loop(0, EBLK // 16, step, pos)

      pos = lax.fori_loop(0, NEB, process_block, jnp.int32(0))
      fire(pos)
      plsc.subcore_barrier()

      wbase = CSL * sid
      for off, n in _WCH:
        pltpu.sync_copy(acc.at[pl.ds(wbase + off, n)], rows.at[pl.ds(0, n)])
        pltpu.sync_copy(rows.at[pl.ds(0, n)],
                        out.at[pl.ds(k * C + wbase + off, n)])
      plsc.subcore_barrier()

  return pl.kernel(body, out_type=out_type, mesh=_mesh(),
                   scratch_types=scratch, name="sc_agg",
                   compiler_params=_SC_PARAMS)


def _agg(tab, gidx, sidx, zrows):
  return _make_agg()(tab, gidx, sidx, zrows)


def _sc_dot(o_s, o_t, e0, e1):
  scratch = [
      pltpu.VMEM((LBLK,), jnp.int32),
      pltpu.VMEM((LBLK,), jnp.int32),
      pltpu.VMEM((LBLK, H), jnp.float32),
      pltpu.VMEM((LBLK, H), jnp.float32),
      pltpu.VMEM((LBLK,), jnp.float32),
      pltpu.SemaphoreType.DMA,
  ]

  def body(os_hbm, ot_hbm, e0_hbm, e1_hbm, out, i0, i1, rs, rt, ob, sem):
    cid = lax.axis_index("c")
    sid = lax.axis_index("s")
    wid = sid * 2 + cid
    base = wid * LPT

    def block(b, _):
      off = base + b * LBLK
      pltpu.sync_copy(e0_hbm.at[pl.ds(off, LBLK)], i0)
      pltpu.sync_copy(e1_hbm.at[pl.ds(off, LBLK)], i1)
      cp0 = pltpu.async_copy(os_hbm.at[i0], rs, sem)
      cp1 = pltpu.async_copy(ot_hbm.at[i1], rt, sem)
      cp0.wait()
      cp1.wait()

      iota = lax.iota(jnp.int32, 16)

      def lab16(jj, _):
        rowi = jj * 16 + iota
        acc = jnp.zeros((16,), jnp.float32)
        for c in range(O):
          ci = jnp.full((16,), c, jnp.int32)
          acc = acc + (plsc.load_gather(rs, [rowi, ci])
                       * plsc.load_gather(rt, [rowi, ci]))
        ob[pl.ds(jj * 16, 16)] = acc
        return 0

      lax.fori_loop(0, LBLK // 16, lab16, 0)
      pltpu.sync_copy(ob, out.at[pl.ds(off, LBLK)])
      return 0

    lax.fori_loop(0, NLB, block, 0)

  f = pl.kernel(body, out_type=jax.ShapeDtypeStruct((LP,), jnp.float32),
                mesh=_mesh(), scratch_types=scratch, name="sc_dot",
                compiler_params=_SC_PARAMS)
  return f(o_s, o_t, e0, e1)


def _tc_xt(target_x, Wlin, blin, Et):
  def body(tx, w, b, et, o):
    o[...] = (jnp.dot(tx[...], w[...], preferred_element_type=jnp.float32)
              + b[...] + et[...])

  return pl.pallas_call(
      body, grid=(NPB,),
      in_specs=[
          pl.BlockSpec((BM, DINP), lambda i: (i, 0)),
          pl.BlockSpec((DINP, H), lambda i: (0, 0)),
          pl.BlockSpec((1, H), lambda i: (0, 0)),
          pl.BlockSpec((BM, H), lambda i: (i, 0)),
      ],
      out_specs=pl.BlockSpec((BM, H), lambda i: (i, 0)),
      out_shape=jax.ShapeDtypeStruct((NP, H), jnp.float32),
  )(target_x, Wlin, blin, Et)


def _tc_combine1(sums, cnt, x, Wl, Wr, b):
  def body(s, c, x_, wl, wr, b_, h_o):
    deg = jnp.clip(c[...], 1.0, None)
    a = s[...] / deg
    h_o[...] = jnp.maximum(
        jnp.dot(a, wl[...], preferred_element_type=jnp.float32)
        + jnp.dot(x_[...], wr[...], preferred_element_type=jnp.float32)
        + b_[...], 0.0)

  return pl.pallas_call(
      body, grid=(NPB,),
      in_specs=[
          pl.BlockSpec((BM, H), lambda i: (i, 0)),
          pl.BlockSpec((BM, 1), lambda i: (i, 0)),
          pl.BlockSpec((BM, H), lambda i: (i, 0)),
          pl.BlockSpec((H, H), lambda i: (0, 0)),
          pl.BlockSpec((H, H), lambda i: (0, 0)),
          pl.BlockSpec((1, H), lambda i: (0, 0)),
      ],
      out_specs=pl.BlockSpec((BM, H), lambda i: (i, 0)),
      out_shape=jax.ShapeDtypeStruct((NP, H), jnp.float32),
  )(sums, cnt, x, Wl, Wr, b)


def _tc_combine2(sums, cnt, h, Wl, Wr, b):
  def body(s, c, h_, wl, wr, b_, o_o):
    deg = jnp.clip(c[...], 1.0, None)
    a = s[...] / deg
    o = (jnp.dot(a, wl[...], preferred_element_type=jnp.float32)
         + jnp.dot(h_[...], wr[...], preferred_element_type=jnp.float32)
         + b_[...])
    o_o[...] = jnp.concatenate([o, jnp.zeros((BM, H - O), jnp.float32)],
                               axis=1)

  return pl.pallas_call(
      body, grid=(NPB,),
      in_specs=[
          pl.BlockSpec((BM, H), lambda i: (i, 0)),
          pl.BlockSpec((BM, 1), lambda i: (i, 0)),
          pl.BlockSpec((BM, H), lambda i: (i, 0)),
          pl.BlockSpec((H, O), lambda i: (0, 0)),
          pl.BlockSpec((H, O), lambda i: (0, 0)),
          pl.BlockSpec((1, O), lambda i: (0, 0)),
      ],
      out_specs=pl.BlockSpec((BM, H), lambda i: (i, 0)),
      out_shape=jax.ShapeDtypeStruct((NP, H), jnp.float32),
  )(sums, cnt, h, Wl, Wr, b)


def kernel(source_node_id, target_node_id, target_x, edge_index,
           edge_label_index, Es, Et, Wlin, blin,
           Wl1_st, b1_st, Wr1_st, Wl1_ts, b1_ts, Wr1_ts,
           Wl2_st, b2_st, Wr2_st, Wl2_ts, b2_ts, Wr2_ts):
  src = edge_index[0].astype(jnp.int32)
  dst = edge_index[1].astype(jnp.int32)
  epad = jnp.full((EP - E,), SENT, jnp.int32)
  src_p = jnp.concatenate([src, epad])
  dst_p = jnp.concatenate([dst, epad])
  src2 = src_p.reshape(EP // 128, 128)
  dst2 = dst_p.reshape(EP // 128, 128)
  e0 = jnp.concatenate([edge_label_index[0].astype(jnp.int32),
                        jnp.zeros((LP - L,), jnp.int32)])
  e1 = jnp.concatenate([edge_label_index[1].astype(jnp.int32),
                        jnp.zeros((LP - L,), jnp.int32)])

  txp = jnp.pad(target_x, ((0, 0), (0, DINP - DIN)))
  wlinp = jnp.pad(Wlin, ((0, DINP - DIN), (0, 0)))
  b_lin = blin.reshape(1, H)
  b1st = b1_st.reshape(1, H)
  b1ts = b1_ts.reshape(1, H)
  b2st = b2_st.reshape(1, O)
  b2ts = b2_ts.reshape(1, O)

  zrows = jnp.zeros((ZSL, H), jnp.float32)
  zd = jnp.zeros((DSL,), jnp.float32)
  onesd = jnp.ones((128,), jnp.float32)

  x_s = Es
  x_t = _tc_xt(txp, wlinp, b_lin, Et)

  cnt_s_p, cnt_t_p = _degrees(src2, dst2, onesd, zd)
  cnt_s2 = (cnt_s_p[:NCT] + cnt_s_p[NCT:])[:, None]
  cnt_t2 = (cnt_t_p[:NCT] + cnt_t_p[NCT:])[:, None]

  sum1_t = _agg(x_s, src_p, dst_p, zrows)
  sum1_s = _agg(x_t, dst_p, src_p, zrows)

  h_t = _tc_combine1(sum1_t, cnt_t2, x_t, Wl1_st, Wr1_st, b1st)
  h_s = _tc_combine1(sum1_s, cnt_s2, x_s, Wl1_ts, Wr1_ts, b1ts)

  sum2_t = _agg(h_s, src_p, dst_p, zrows)
  sum2_s = _agg(h_t, dst_p, src_p, zrows)

  o_t = _tc_combine2(sum2_t, cnt_t2, h_t, Wl2_st, Wr2_st, b2st)
  o_s = _tc_combine2(sum2_s, cnt_s2, h_s, Wl2_ts, Wr2_ts, b2ts)

  return _sc_dot(o_s, o_t, e0, e1)[:L]
